# baseline (speedup 1.0000x reference)
import dataclasses
import functools

import jax
import jax.numpy as jnp
from jax import lax
from jax.experimental import pallas as pl
from jax.experimental.pallas import tpu as pltpu
from jax.experimental.pallas import tpu_sc as plsc

CUT = 5.0
_P = 6
_EA = -(_P + 1) * (_P + 2) / 2.0
_EB = _P * (_P + 2) * 1.0
_EC = -_P * (_P + 1) / 2.0

NW = 32
CH = 80
HI = jax.lax.Precision.HIGHEST

_MESH = dict(core_axis_name="c", subcore_axis_name="s")

_SC_CP = pltpu.CompilerParams()
if "needs_layout_passes" in pltpu.CompilerParams.__dataclass_fields__:
    _SC_CP = dataclasses.replace(_SC_CP, needs_layout_passes=False)


def _row_slab(sid, n):
    big = (n // (16 * 8)) * 8
    start = pl.multiple_of(sid * big, 8)
    return start, big, n - 15 * big


def _acc_zero(z_h, acc, sid, n):
    start, big, last = _row_slab(sid, n)
    @pl.when(sid < 15)
    def _():
        pltpu.sync_copy(z_h.at[pl.ds(start, big)], acc.at[pl.ds(start, big)])
    @pl.when(sid == 15)
    def _():
        pltpu.sync_copy(z_h.at[pl.ds(15 * big, last)],
                        acc.at[pl.ds(15 * big, last)])


def _acc_dump(acc, out_h, cid, sid, n):
    start, big, last = _row_slab(sid, n)
    @pl.when(sid < 15)
    def _():
        pltpu.sync_copy(acc.at[pl.ds(start, big)],
                        out_h.at[cid, pl.ds(start, big)])
    @pl.when(sid == 15)
    def _():
        pltpu.sync_copy(acc.at[pl.ds(15 * big, last)],
                        out_h.at[cid, pl.ds(15 * big, last)])


def _sc_edge_d2(posx, posy, posz, row, col):
    n = posx.shape[0]
    e = row.shape[0]
    epw = e // NW

    @functools.partial(
        pl.kernel,
        out_type=jax.ShapeDtypeStruct((e,), jnp.float32),
        mesh=plsc.VectorSubcoreMesh(**_MESH),
        compiler_params=_SC_CP,
        scratch_types=[
            pltpu.VMEM((n,), jnp.float32),
            pltpu.VMEM((n,), jnp.float32),
            pltpu.VMEM((n,), jnp.float32),
            pltpu.VMEM((epw,), jnp.int32),
            pltpu.VMEM((epw,), jnp.int32),
            pltpu.VMEM((epw,), jnp.float32),
        ],
    )
    def k(px_h, py_h, pz_h, row_h, col_h, out_h, px, py, pz, rv, cv, dv):
        cid = lax.axis_index("c")
        sid = lax.axis_index("s")
        base = pl.multiple_of((cid * 16 + sid) * epw, 8)
        pltpu.sync_copy(px_h, px)
        pltpu.sync_copy(py_h, py)
        pltpu.sync_copy(pz_h, pz)
        pltpu.sync_copy(row_h.at[pl.ds(base, epw)], rv)
        pltpu.sync_copy(col_h.at[pl.ds(base, epw)], cv)

        @pl.loop(0, epw, step=16)
        def _(i):
            r = rv[pl.ds(i, 16)]
            c = cv[pl.ds(i, 16)]
            ax = plsc.load_gather(px, [r]) - plsc.load_gather(px, [c])
            ay = plsc.load_gather(py, [r]) - plsc.load_gather(py, [c])
            az = plsc.load_gather(pz, [r]) - plsc.load_gather(pz, [c])
            dv[pl.ds(i, 16)] = ax * ax + ay * ay + az * az + 1e-12

        pltpu.sync_copy(dv, out_h.at[pl.ds(base, epw)])

    return k(posx, posy, posz, row, col)


def _sc_scatter(vals, col, zeros):
    n, d = zeros.shape
    e = col.shape[0]
    epw = e // NW
    nch = epw // CH
    nps = n // 16

    @functools.partial(
        pl.kernel,
        out_type=jax.ShapeDtypeStruct((2, n, d), jnp.float32),
        mesh=plsc.VectorSubcoreMesh(**_MESH),
        scratch_types=[
            pltpu.VMEM_SHARED((n, d), jnp.float32),
            pltpu.VMEM((2, CH), jnp.int32),
            pltpu.VMEM((2, CH, d), jnp.float32),
            pltpu.SemaphoreType.DMA((2,)),
            pltpu.SemaphoreType.DMA((2,)),
        ],
    )
    def k(v_h, col_h, z_h, out_h, acc, cv, vv, s_c, s_v):
        cid = lax.axis_index("c")
        sid = lax.axis_index("s")
        base = (cid * 16 + sid) * epw
        _acc_zero(z_h, acc, sid, n)
        plsc.subcore_barrier()

        def issue_pre(j, b):
            off = pl.multiple_of(base + j * CH, 8)
            pltpu.async_copy(col_h.at[pl.ds(off, CH)], cv.at[b], s_c.at[b])
            pltpu.async_copy(v_h.at[pl.ds(off, CH)], vv.at[b], s_v.at[b])

        issue_pre(0, 0)
        issue_pre(1, 1)

        @pl.loop(0, nch + 1, step=2)
        def _(jj):
            for b in (0, 1):
                j = jj + b

                @pl.when(j < nch)
                def _():
                    pltpu.make_async_copy(v_h.at[pl.ds(0, CH)], vv.at[b],
                                          s_v.at[b]).wait()
                    pltpu.make_async_copy(col_h.at[pl.ds(0, CH)], cv.at[b],
                                          s_c.at[b]).wait()
                    pltpu.sync_copy(vv.at[b], acc.at[cv.at[b]], add=True)

                    @pl.when(j + 2 < nch)
                    def _():
                        issue_pre(j + 2, b)

        plsc.subcore_barrier()
        _acc_dump(acc, out_h, cid, sid, n)

    return k(vals, col, zeros)


def _sc_gms(table, mult, row, col, zeros):
    n, d = table.shape
    e = row.shape[0]
    epw = e // NW
    chg = 64
    nch = epw // chg
    rem = epw - nch * chg

    @functools.partial(
        pl.kernel,
        out_type=jax.ShapeDtypeStruct((2, n, d), jnp.float32),
        mesh=plsc.VectorSubcoreMesh(**_MESH),
        scratch_types=[
            pltpu.VMEM_SHARED((n, d), jnp.float32),
            pltpu.VMEM((2, chg), jnp.int32),
            pltpu.VMEM((4, chg), jnp.int32),
            pltpu.VMEM((3, chg, d), jnp.float32),
            pltpu.VMEM((2, chg, d), jnp.float32),
            pltpu.VMEM((rem,), jnp.int32),
            pltpu.VMEM((rem,), jnp.int32),
            pltpu.VMEM((rem, d), jnp.float32),
            pltpu.VMEM((rem, d), jnp.float32),
            pltpu.SemaphoreType.DMA((2,)),
            pltpu.SemaphoreType.DMA((4,)),
            pltpu.SemaphoreType.DMA((2,)),
            pltpu.SemaphoreType.DMA((3,)),
            pltpu.SemaphoreType.DMA((3,)),
        ],
    )
    def k(t_h, m_h, row_h, col_h, z_h, out_h, acc, rv, cv, gv, mv,
          rvt, cvt, gvt, mvt, s_r, s_c, s_m, s_g, s_s):
        cid = lax.axis_index("c")
        sid = lax.axis_index("s")
        base = (cid * 16 + sid) * epw
        _acc_zero(z_h, acc, sid, n)
        plsc.subcore_barrier()

        def issue_pre(j, u):
            off = pl.multiple_of(base + j * chg, 8)
            pltpu.async_copy(row_h.at[pl.ds(off, chg)], rv.at[u % 2],
                             s_r.at[u % 2])
            pltpu.async_copy(col_h.at[pl.ds(off, chg)], cv.at[u % 4],
                             s_c.at[u % 4])
            pltpu.async_copy(m_h.at[pl.ds(off, chg)], mv.at[u % 2],
                             s_m.at[u % 2])

        def issue_gather(u):
            pltpu.make_async_copy(row_h.at[pl.ds(0, chg)], rv.at[u % 2],
                                  s_r.at[u % 2]).wait()
            pltpu.async_copy(t_h.at[rv.at[u % 2]], gv.at[u % 3],
                             s_g.at[u % 3])

        issue_pre(0, 0)
        issue_pre(1, 1)
        issue_gather(0)
        issue_gather(1)

        @pl.loop(0, nch + 11, step=12)
        def _(jj):
            for u in range(12):
                j = jj + u
                u2, u3, u4 = u % 2, u % 3, u % 4

                @pl.when(j < nch)
                def _():
                    @pl.when(j >= 2)
                    def _():
                        pltpu.make_async_copy(
                            gv.at[(u + 1) % 3],
                            acc.at[cv.at[(u + 2) % 4]],
                            s_s.at[(u + 1) % 3]).wait()

                    pltpu.make_async_copy(t_h.at[rv.at[u2]], gv.at[u3],
                                          s_g.at[u3]).wait()
                    pltpu.make_async_copy(m_h.at[pl.ds(0, chg)], mv.at[u2],
                                          s_m.at[u2]).wait()
                    pltpu.make_async_copy(col_h.at[pl.ds(0, chg)], cv.at[u4],
                                          s_c.at[u4]).wait()

                    @pl.when(jnp.logical_and(j >= 1, j + 1 < nch))
                    def _():
                        issue_gather(u + 1)

                    @pl.loop(0, chg, step=2)
                    def _(i):
                        for r in range(2):
                            for h in range(d // 16):
                                sl = pl.ds(h * 16, 16)
                                gv[u3, i + r, sl] = \
                                    gv[u3, i + r, sl] * mv[u2, i + r, sl]

                    pltpu.async_copy(gv.at[u3], acc.at[cv.at[u4]],
                                     s_s.at[u3], add=True)

                    @pl.when(j + 2 < nch)
                    def _():
                        issue_pre(j + 2, u + 2)

        for jlast in (nch - 2, nch - 1):
            pltpu.make_async_copy(gv.at[jlast % 3],
                                  acc.at[cv.at[jlast % 4]],
                                  s_s.at[jlast % 3]).wait()

        toff = pl.multiple_of(base + nch * chg, 8)
        pltpu.sync_copy(row_h.at[pl.ds(toff, rem)], rvt)
        pltpu.sync_copy(col_h.at[pl.ds(toff, rem)], cvt)
        pltpu.sync_copy(m_h.at[pl.ds(toff, rem)], mvt)
        pltpu.async_copy(t_h.at[rvt], gvt, s_g.at[0]).wait()

        @pl.loop(0, rem)
        def _(i):
            for h in range(d // 16):
                sl = pl.ds(h * 16, 16)
                gvt[i, sl] = gvt[i, sl] * mvt[i, sl]

        pltpu.sync_copy(gvt, acc.at[cvt], add=True)

        plsc.subcore_barrier()
        _acc_dump(acc, out_h, cid, sid, n)

    return k(table, mult, row, col, zeros)


def _tc_multipliers(d2c, freq, emb_W1, emb_b1r, emb_W2, emb_b2r,
                    int_rbf1_W, int_rbf2_W, out_rbf_W):
    e = d2c.shape[0]
    be = 2560
    nb = out_rbf_W.shape[0] - 1

    def rbf_of(d2_blk, f_blk):
        dd = jnp.sqrt(d2_blk) * (1.0 / CUT)
        inv = 1.0 / dd
        dd2 = dd * dd
        dd4 = dd2 * dd2
        dd5 = dd4 * dd
        env = inv + _EA * dd5 + _EB * dd5 * dd + _EC * dd5 * dd2
        return env * jnp.sin(dd * f_blk)

    def body_e(d2_ref, f_ref, w1_ref, b1_ref, w2_ref, b2_ref, e_ref):
        rbf = rbf_of(d2_ref[:], f_ref[:])
        u = jax.nn.silu(jnp.dot(rbf, w1_ref[:]) + b1_ref[:])
        e_ref[...] = jnp.dot(u, w2_ref[:]) + b2_ref[:]

    def body_r(d2_ref, f_ref, r1_ref, r2_ref, orw_ref, *outs):
        rbf = rbf_of(d2_ref[:], f_ref[:])
        for i in range(nb + 1):
            outs[i][...] = jnp.dot(rbf, orw_ref[i])
        for i in range(nb):
            t = jax.nn.silu(jnp.dot(rbf, r1_ref[i]))
            outs[nb + 1 + i][...] = jnp.dot(t, r2_ref[i])

    full = lambda arr: pl.BlockSpec(arr.shape, lambda j: (0,) * arr.ndim)
    eb = pl.BlockSpec((be, 128), lambda j: (j, 0))
    d2b = pl.BlockSpec((be, 1), lambda j: (j, 0))
    e128 = pl.pallas_call(
        body_e,
        grid=(e // be,),
        in_specs=[d2b, full(freq), full(emb_W1), full(emb_b1r),
                  full(emb_W2), full(emb_b2r)],
        out_specs=eb,
        out_shape=jax.ShapeDtypeStruct((e, 128), jnp.float32),
    )(d2c, freq, emb_W1, emb_b1r, emb_W2, emb_b2r)
    rs = pl.pallas_call(
        body_r,
        grid=(e // be,),
        in_specs=[d2b, full(freq), full(int_rbf1_W), full(int_rbf2_W),
                  full(out_rbf_W)],
        out_specs=[eb] * 9,
        out_shape=[jax.ShapeDtypeStruct((e, 128), jnp.float32)] * 9,
    )(d2c, freq, int_rbf1_W, int_rbf2_W, out_rbf_W)
    return [e128] + list(rs)


def _tc_init(zc, emb_pad, accE, upW, upbr):
    n = zc.shape[0]
    bn = 1000

    def body(z_ref, e_ref, a_ref, uw_ref, ub_ref, x_ref, xup_ref):
        oh = (z_ref[:] == lax.broadcasted_iota(jnp.int32, (bn, 128), 1))
        x = jnp.dot(oh.astype(jnp.float32), e_ref[:], precision=HI)
        x = x + a_ref[0] + a_ref[1]
        x_ref[...] = x
        xup_ref[...] = jnp.dot(x, uw_ref[:]) + ub_ref[:]

    full = lambda arr: pl.BlockSpec(arr.shape, lambda j: (0,) * arr.ndim)
    nb128 = pl.BlockSpec((bn, 128), lambda j: (j, 0))
    return pl.pallas_call(
        body,
        grid=(n // bn,),
        in_specs=[pl.BlockSpec((bn, 1), lambda j: (j, 0)), full(emb_pad),
                  pl.BlockSpec((2, bn, 128), lambda j: (0, j, 0)),
                  full(upW), full(upbr)],
        out_specs=[nb128, nb128],
        out_shape=[jax.ShapeDtypeStruct((n, 128), jnp.float32)] * 2,
    )(zc, emb_pad, accE, upW, upbr)


def _tc_round(x, aggO, aggX, P_in, W1, b1r, W2, b2r, upW, upbr):
    n = x.shape[0]
    bn = 1000

    def body(x_ref, aO_ref, aX_ref, p_ref, w1_ref, b1_ref, w2_ref, b2_ref,
             uw_ref, ub_ref, pout_ref, xn_ref, xup_ref):
        agg = aO_ref[0] + aO_ref[1]
        t = jax.nn.silu(jnp.dot(agg, w1_ref[:]) + b1_ref[:])
        pout_ref[...] = p_ref[:] + jnp.dot(t, w2_ref[:]) + b2_ref[:]
        xn = x_ref[:] + aX_ref[0] + aX_ref[1]
        xn_ref[...] = xn
        xup_ref[...] = jnp.dot(xn, uw_ref[:]) + ub_ref[:]

    full = lambda arr: pl.BlockSpec(arr.shape, lambda j: (0,) * arr.ndim)
    nb128 = pl.BlockSpec((bn, 128), lambda j: (j, 0))
    nb1 = pl.BlockSpec((bn, 1), lambda j: (j, 0))
    agg_spec = pl.BlockSpec((2, bn, 128), lambda j: (0, j, 0))
    return pl.pallas_call(
        body,
        grid=(n // bn,),
        in_specs=[nb128, agg_spec, agg_spec, nb1, full(W1), full(b1r),
                  full(W2), full(b2r), full(upW), full(upbr)],
        out_specs=[nb1, nb128, nb128],
        out_shape=[jax.ShapeDtypeStruct((n, 1), jnp.float32),
                   jax.ShapeDtypeStruct((n, 128), jnp.float32),
                   jax.ShapeDtypeStruct((n, 128), jnp.float32)],
    )(x, aggO, aggX, P_in, W1, b1r, W2, b2r, upW, upbr)


def _tc_final(aggO, P_in, W1, b1r, W2, b2r):
    n = P_in.shape[0]
    bn = 1000

    def body(aO_ref, p_ref, w1_ref, b1_ref, w2_ref, b2_ref, pout_ref):
        agg = aO_ref[0] + aO_ref[1]
        t = jax.nn.silu(jnp.dot(agg, w1_ref[:]) + b1_ref[:])
        pout_ref[...] = p_ref[:] + jnp.dot(t, w2_ref[:]) + b2_ref[:]

    full = lambda arr: pl.BlockSpec(arr.shape, lambda j: (0,) * arr.ndim)
    return pl.pallas_call(
        body,
        grid=(n // bn,),
        in_specs=[pl.BlockSpec((2, bn, 128), lambda j: (0, j, 0)),
                  pl.BlockSpec((bn, 1), lambda j: (j, 0)), full(W1),
                  full(b1r), full(W2), full(b2r)],
        out_specs=pl.BlockSpec((bn, 1), lambda j: (j, 0)),
        out_shape=jax.ShapeDtypeStruct((n, 1), jnp.float32),
    )(aggO, P_in, W1, b1r, W2, b2r)


def kernel(z, pos, edge_index, atom_emb_W, rbf_freq, emb_W1, emb_b1, emb_W2,
           emb_b2, int_rbf1_W, int_rbf2_W, int_up_W, int_up_b, out_rbf_W,
           out_mlp_W1, out_mlp_b1, out_mlp_W2, out_mlp_b2):
    n = pos.shape[0]
    nb = int_rbf1_W.shape[0]
    h = atom_emb_W.shape[1]

    row = edge_index[0].astype(jnp.int32)
    col = edge_index[1].astype(jnp.int32)
    posx = jnp.asarray(pos[:, 0])
    posy = jnp.asarray(pos[:, 1])
    posz = jnp.asarray(pos[:, 2])

    d2 = _sc_edge_d2(posx, posy, posz, row, col)

    mults = _tc_multipliers(d2[:, None], rbf_freq[None, :], emb_W1,
                            emb_b1[None, :], emb_W2, emb_b2[None, :],
                            int_rbf1_W, int_rbf2_W, out_rbf_W)
    e128 = mults[0]
    Rout = mults[1:2 + nb]
    Rint = mults[2 + nb:]

    zeros128 = jnp.zeros((n, 128), jnp.float32)

    accE = _sc_scatter(e128, col, zeros128)

    emb_pad = jnp.zeros((128, h), jnp.float32).at[:atom_emb_W.shape[0]].set(
        atom_emb_W)
    x, xup = _tc_init(z.astype(jnp.int32)[:, None], emb_pad, accE,
                      int_up_W[0], int_up_b[0][None, :])

    P = jnp.zeros((n, 1), jnp.float32)
    for r in range(nb):
        aggO = _sc_gms(x, Rout[r], row, col, zeros128)
        aggX = _sc_gms(xup, Rint[r], row, col, zeros128)
        upW = int_up_W[r + 1] if r + 1 < nb else int_up_W[r]
        upb = int_up_b[r + 1] if r + 1 < nb else int_up_b[r]
        P, x, xup = _tc_round(x, aggO, aggX, P, out_mlp_W1[r],
                              out_mlp_b1[r][None, :], out_mlp_W2[r],
                              out_mlp_b2[r][None, :], upW, upb[None, :])

    aggO = _sc_gms(x, Rout[nb], row, col, zeros128)
    P = _tc_final(aggO, P, out_mlp_W1[nb], out_mlp_b1[nb][None, :],
                  out_mlp_W2[nb], out_mlp_b2[nb][None, :])
    return P


# speedup vs baseline: 1.1668x; 1.1668x over previous
import dataclasses
import functools

import jax
import jax.numpy as jnp
from jax import lax
from jax.experimental import pallas as pl
from jax.experimental.pallas import tpu as pltpu
from jax.experimental.pallas import tpu_sc as plsc

CUT = 5.0
_P = 6
_EA = -(_P + 1) * (_P + 2) / 2.0
_EB = _P * (_P + 2) * 1.0
_EC = -_P * (_P + 1) / 2.0

NW = 32
CH = 80
HI = jax.lax.Precision.HIGHEST

_MESH = dict(core_axis_name="c", subcore_axis_name="s")

_SC_CP = pltpu.CompilerParams()
if "needs_layout_passes" in pltpu.CompilerParams.__dataclass_fields__:
    _SC_CP = dataclasses.replace(_SC_CP, needs_layout_passes=False)


def _row_slab(sid, n):
    big = (n // (16 * 8)) * 8
    start = pl.multiple_of(sid * big, 8)
    return start, big, n - 15 * big


def _acc_zero(z_h, acc, sid, n):
    start, big, last = _row_slab(sid, n)
    @pl.when(sid < 15)
    def _():
        pltpu.sync_copy(z_h.at[pl.ds(start, big)], acc.at[pl.ds(start, big)])
    @pl.when(sid == 15)
    def _():
        pltpu.sync_copy(z_h.at[pl.ds(15 * big, last)],
                        acc.at[pl.ds(15 * big, last)])


def _acc_dump(acc, out_h, cid, sid, n):
    start, big, last = _row_slab(sid, n)
    @pl.when(sid < 15)
    def _():
        pltpu.sync_copy(acc.at[pl.ds(start, big)],
                        out_h.at[cid, pl.ds(start, big)])
    @pl.when(sid == 15)
    def _():
        pltpu.sync_copy(acc.at[pl.ds(15 * big, last)],
                        out_h.at[cid, pl.ds(15 * big, last)])


def _sc_edge_d2(posx, posy, posz, row, col):
    n = posx.shape[0]
    e = row.shape[0]
    epw = e // NW

    @functools.partial(
        pl.kernel,
        out_type=jax.ShapeDtypeStruct((e,), jnp.float32),
        mesh=plsc.VectorSubcoreMesh(**_MESH),
        compiler_params=_SC_CP,
        scratch_types=[
            pltpu.VMEM((n,), jnp.float32),
            pltpu.VMEM((n,), jnp.float32),
            pltpu.VMEM((n,), jnp.float32),
            pltpu.VMEM((epw,), jnp.int32),
            pltpu.VMEM((epw,), jnp.int32),
            pltpu.VMEM((epw,), jnp.float32),
        ],
    )
    def k(px_h, py_h, pz_h, row_h, col_h, out_h, px, py, pz, rv, cv, dv):
        cid = lax.axis_index("c")
        sid = lax.axis_index("s")
        base = pl.multiple_of((cid * 16 + sid) * epw, 8)
        pltpu.sync_copy(px_h, px)
        pltpu.sync_copy(py_h, py)
        pltpu.sync_copy(pz_h, pz)
        pltpu.sync_copy(row_h.at[pl.ds(base, epw)], rv)
        pltpu.sync_copy(col_h.at[pl.ds(base, epw)], cv)

        @pl.loop(0, epw, step=16)
        def _(i):
            r = rv[pl.ds(i, 16)]
            c = cv[pl.ds(i, 16)]
            ax = plsc.load_gather(px, [r]) - plsc.load_gather(px, [c])
            ay = plsc.load_gather(py, [r]) - plsc.load_gather(py, [c])
            az = plsc.load_gather(pz, [r]) - plsc.load_gather(pz, [c])
            dv[pl.ds(i, 16)] = ax * ax + ay * ay + az * az + 1e-12

        pltpu.sync_copy(dv, out_h.at[pl.ds(base, epw)])

    return k(posx, posy, posz, row, col)


def _sc_scatter(vals, col, zeros):
    n, d = zeros.shape
    e = col.shape[0]
    epw = e // NW
    nch = epw // CH
    nps = n // 16

    @functools.partial(
        pl.kernel,
        out_type=jax.ShapeDtypeStruct((2, n, d), jnp.float32),
        mesh=plsc.VectorSubcoreMesh(**_MESH),
        scratch_types=[
            pltpu.VMEM_SHARED((n, d), jnp.float32),
            pltpu.VMEM((2, CH), jnp.int32),
            pltpu.VMEM((2, CH, d), jnp.float32),
            pltpu.SemaphoreType.DMA((2,)),
            pltpu.SemaphoreType.DMA((2,)),
        ],
    )
    def k(v_h, col_h, z_h, out_h, acc, cv, vv, s_c, s_v):
        cid = lax.axis_index("c")
        sid = lax.axis_index("s")
        base = (cid * 16 + sid) * epw
        _acc_zero(z_h, acc, sid, n)
        plsc.subcore_barrier()

        def issue_pre(j, b):
            off = pl.multiple_of(base + j * CH, 8)
            pltpu.async_copy(col_h.at[pl.ds(off, CH)], cv.at[b], s_c.at[b])
            pltpu.async_copy(v_h.at[pl.ds(off, CH)], vv.at[b], s_v.at[b])

        issue_pre(0, 0)
        issue_pre(1, 1)

        @pl.loop(0, nch + 1, step=2)
        def _(jj):
            for b in (0, 1):
                j = jj + b

                @pl.when(j < nch)
                def _():
                    pltpu.make_async_copy(v_h.at[pl.ds(0, CH)], vv.at[b],
                                          s_v.at[b]).wait()
                    pltpu.make_async_copy(col_h.at[pl.ds(0, CH)], cv.at[b],
                                          s_c.at[b]).wait()
                    pltpu.sync_copy(vv.at[b], acc.at[cv.at[b]], add=True)

                    @pl.when(j + 2 < nch)
                    def _():
                        issue_pre(j + 2, b)

        plsc.subcore_barrier()
        _acc_dump(acc, out_h, cid, sid, n)

    return k(vals, col, zeros)


def _sc_gms(table, mult, row, col, zeros):
    n, d = table.shape
    e = row.shape[0]
    epw = e // NW
    chg = 64
    nch = epw // chg
    rem = epw - nch * chg

    @functools.partial(
        pl.kernel,
        out_type=jax.ShapeDtypeStruct((2, n, d), jnp.float32),
        mesh=plsc.VectorSubcoreMesh(**_MESH),
        scratch_types=[
            pltpu.VMEM_SHARED((n, d), jnp.float32),
            pltpu.VMEM((2, chg), jnp.int32),
            pltpu.VMEM((4, chg), jnp.int32),
            pltpu.VMEM((3, chg, d), jnp.float32),
            pltpu.VMEM((2, chg, d), jnp.float32),
            pltpu.VMEM((rem,), jnp.int32),
            pltpu.VMEM((rem,), jnp.int32),
            pltpu.VMEM((rem, d), jnp.float32),
            pltpu.VMEM((rem, d), jnp.float32),
            pltpu.SemaphoreType.DMA((2,)),
            pltpu.SemaphoreType.DMA((4,)),
            pltpu.SemaphoreType.DMA((2,)),
            pltpu.SemaphoreType.DMA((3,)),
            pltpu.SemaphoreType.DMA((3,)),
        ],
    )
    def k(t_h, m_h, row_h, col_h, z_h, out_h, acc, rv, cv, gv, mv,
          rvt, cvt, gvt, mvt, s_r, s_c, s_m, s_g, s_s):
        cid = lax.axis_index("c")
        sid = lax.axis_index("s")
        base = (cid * 16 + sid) * epw
        _acc_zero(z_h, acc, sid, n)
        plsc.subcore_barrier()

        def issue_pre(j, u):
            off = pl.multiple_of(base + j * chg, 8)
            pltpu.async_copy(row_h.at[pl.ds(off, chg)], rv.at[u % 2],
                             s_r.at[u % 2])
            pltpu.async_copy(col_h.at[pl.ds(off, chg)], cv.at[u % 4],
                             s_c.at[u % 4])
            pltpu.async_copy(m_h.at[pl.ds(off, chg)], mv.at[u % 2],
                             s_m.at[u % 2])

        def issue_gather(u):
            pltpu.make_async_copy(row_h.at[pl.ds(0, chg)], rv.at[u % 2],
                                  s_r.at[u % 2]).wait()
            pltpu.async_copy(t_h.at[rv.at[u % 2]], gv.at[u % 3],
                             s_g.at[u % 3])

        issue_pre(0, 0)
        issue_pre(1, 1)
        issue_gather(0)
        issue_gather(1)

        @pl.loop(0, nch + 11, step=12)
        def _(jj):
            for u in range(12):
                j = jj + u
                u2, u3, u4 = u % 2, u % 3, u % 4

                @pl.when(j < nch)
                def _():
                    @pl.when(j >= 2)
                    def _():
                        pltpu.make_async_copy(
                            gv.at[(u + 1) % 3],
                            acc.at[cv.at[(u + 2) % 4]],
                            s_s.at[(u + 1) % 3]).wait()

                    pltpu.make_async_copy(t_h.at[rv.at[u2]], gv.at[u3],
                                          s_g.at[u3]).wait()
                    pltpu.make_async_copy(m_h.at[pl.ds(0, chg)], mv.at[u2],
                                          s_m.at[u2]).wait()
                    pltpu.make_async_copy(col_h.at[pl.ds(0, chg)], cv.at[u4],
                                          s_c.at[u4]).wait()

                    @pl.when(jnp.logical_and(j >= 1, j + 1 < nch))
                    def _():
                        issue_gather(u + 1)

                    @pl.loop(0, chg, step=2)
                    def _(i):
                        for r in range(2):
                            for h in range(d // 16):
                                sl = pl.ds(h * 16, 16)
                                gv[u3, i + r, sl] = \
                                    gv[u3, i + r, sl] * mv[u2, i + r, sl]

                    pltpu.async_copy(gv.at[u3], acc.at[cv.at[u4]],
                                     s_s.at[u3], add=True)

                    @pl.when(j + 2 < nch)
                    def _():
                        issue_pre(j + 2, u + 2)

        for jlast in (nch - 2, nch - 1):
            pltpu.make_async_copy(gv.at[jlast % 3],
                                  acc.at[cv.at[jlast % 4]],
                                  s_s.at[jlast % 3]).wait()

        toff = pl.multiple_of(base + nch * chg, 8)
        pltpu.sync_copy(row_h.at[pl.ds(toff, rem)], rvt)
        pltpu.sync_copy(col_h.at[pl.ds(toff, rem)], cvt)
        pltpu.sync_copy(m_h.at[pl.ds(toff, rem)], mvt)
        pltpu.async_copy(t_h.at[rvt], gvt, s_g.at[0]).wait()

        @pl.loop(0, rem)
        def _(i):
            for h in range(d // 16):
                sl = pl.ds(h * 16, 16)
                gvt[i, sl] = gvt[i, sl] * mvt[i, sl]

        pltpu.sync_copy(gvt, acc.at[cvt], add=True)

        plsc.subcore_barrier()
        _acc_dump(acc, out_h, cid, sid, n)

    return k(table, mult, row, col, zeros)


def _tc_multipliers(d2c, freq, emb_W1, emb_b1r, emb_W2, emb_b2r,
                    int_rbf1_W, int_rbf2_W, out_rbf_W):
    e = d2c.shape[0]
    be = 2560
    nb = out_rbf_W.shape[0] - 1

    def rbf_of(d2_blk, f_blk):
        dd = jnp.sqrt(d2_blk) * (1.0 / CUT)
        inv = 1.0 / dd
        dd2 = dd * dd
        dd4 = dd2 * dd2
        dd5 = dd4 * dd
        env = inv + _EA * dd5 + _EB * dd5 * dd + _EC * dd5 * dd2
        return env * jnp.sin(dd * f_blk)

    def body(d2_ref, f_ref, w1_ref, b1_ref, w2_ref, b2_ref, r1_ref, r2_ref,
             orw_ref, *outs):
        rbf = rbf_of(d2_ref[:], f_ref[:])
        u = jax.nn.silu(jnp.dot(rbf, w1_ref[:]) + b1_ref[:])
        outs[0][...] = jnp.dot(u, w2_ref[:]) + b2_ref[:]
        for i in range(nb + 1):
            outs[1 + i][...] = jnp.dot(rbf, orw_ref[i])
        for i in range(nb):
            t = jax.nn.silu(jnp.dot(rbf, r1_ref[i]))
            outs[6 + i][...] = jnp.dot(t, r2_ref[i])

    full = lambda arr: pl.BlockSpec(arr.shape, lambda j: (0,) * arr.ndim)
    eb = pl.BlockSpec((be, 128), lambda j: (j, 0))
    d2b = pl.BlockSpec((be, 1), lambda j: (j, 0))
    return pl.pallas_call(
        body,
        grid=(e // be,),
        in_specs=[d2b, full(freq), full(emb_W1), full(emb_b1r),
                  full(emb_W2), full(emb_b2r), full(int_rbf1_W),
                  full(int_rbf2_W), full(out_rbf_W)],
        out_specs=[eb] * 10,
        out_shape=[jax.ShapeDtypeStruct((e, 128), jnp.float32)] * 10,
    )(d2c, freq, emb_W1, emb_b1r, emb_W2, emb_b2r, int_rbf1_W, int_rbf2_W,
      out_rbf_W)


def _tc_init(zc, emb_pad, accE, upW, upbr):
    n = zc.shape[0]
    bn = 1000

    def body(z_ref, e_ref, a_ref, uw_ref, ub_ref, x_ref, xup_ref):
        oh = (z_ref[:] == lax.broadcasted_iota(jnp.int32, (bn, 128), 1))
        x = jnp.dot(oh.astype(jnp.float32), e_ref[:], precision=HI)
        x = x + a_ref[0] + a_ref[1]
        x_ref[...] = x
        xup_ref[...] = jnp.dot(x, uw_ref[:]) + ub_ref[:]

    full = lambda arr: pl.BlockSpec(arr.shape, lambda j: (0,) * arr.ndim)
    nb128 = pl.BlockSpec((bn, 128), lambda j: (j, 0))
    return pl.pallas_call(
        body,
        grid=(n // bn,),
        in_specs=[pl.BlockSpec((bn, 1), lambda j: (j, 0)), full(emb_pad),
                  pl.BlockSpec((2, bn, 128), lambda j: (0, j, 0)),
                  full(upW), full(upbr)],
        out_specs=[nb128, nb128],
        out_shape=[jax.ShapeDtypeStruct((n, 128), jnp.float32)] * 2,
    )(zc, emb_pad, accE, upW, upbr)


def _tc_round(x, aggO, aggX, P_in, W1, b1r, W2, b2r, upW, upbr):
    n = x.shape[0]
    bn = 1000

    def body(x_ref, aO_ref, aX_ref, p_ref, w1_ref, b1_ref, w2_ref, b2_ref,
             uw_ref, ub_ref, pout_ref, xn_ref, xup_ref):
        agg = aO_ref[0] + aO_ref[1]
        t = jax.nn.silu(jnp.dot(agg, w1_ref[:]) + b1_ref[:])
        pout_ref[...] = p_ref[:] + jnp.dot(t, w2_ref[:]) + b2_ref[:]
        xn = x_ref[:] + aX_ref[0] + aX_ref[1]
        xn_ref[...] = xn
        xup_ref[...] = jnp.dot(xn, uw_ref[:]) + ub_ref[:]

    full = lambda arr: pl.BlockSpec(arr.shape, lambda j: (0,) * arr.ndim)
    nb128 = pl.BlockSpec((bn, 128), lambda j: (j, 0))
    nb1 = pl.BlockSpec((bn, 1), lambda j: (j, 0))
    agg_spec = pl.BlockSpec((2, bn, 128), lambda j: (0, j, 0))
    return pl.pallas_call(
        body,
        grid=(n // bn,),
        in_specs=[nb128, agg_spec, agg_spec, nb1, full(W1), full(b1r),
                  full(W2), full(b2r), full(upW), full(upbr)],
        out_specs=[nb1, nb128, nb128],
        out_shape=[jax.ShapeDtypeStruct((n, 1), jnp.float32),
                   jax.ShapeDtypeStruct((n, 128), jnp.float32),
                   jax.ShapeDtypeStruct((n, 128), jnp.float32)],
    )(x, aggO, aggX, P_in, W1, b1r, W2, b2r, upW, upbr)


def _tc_final(aggO, P_in, W1, b1r, W2, b2r):
    n = P_in.shape[0]
    bn = 1000

    def body(aO_ref, p_ref, w1_ref, b1_ref, w2_ref, b2_ref, pout_ref):
        agg = aO_ref[0] + aO_ref[1]
        t = jax.nn.silu(jnp.dot(agg, w1_ref[:]) + b1_ref[:])
        pout_ref[...] = p_ref[:] + jnp.dot(t, w2_ref[:]) + b2_ref[:]

    full = lambda arr: pl.BlockSpec(arr.shape, lambda j: (0,) * arr.ndim)
    return pl.pallas_call(
        body,
        grid=(n // bn,),
        in_specs=[pl.BlockSpec((2, bn, 128), lambda j: (0, j, 0)),
                  pl.BlockSpec((bn, 1), lambda j: (j, 0)), full(W1),
                  full(b1r), full(W2), full(b2r)],
        out_specs=pl.BlockSpec((bn, 1), lambda j: (j, 0)),
        out_shape=jax.ShapeDtypeStruct((n, 1), jnp.float32),
    )(aggO, P_in, W1, b1r, W2, b2r)


def kernel(z, pos, edge_index, atom_emb_W, rbf_freq, emb_W1, emb_b1, emb_W2,
           emb_b2, int_rbf1_W, int_rbf2_W, int_up_W, int_up_b, out_rbf_W,
           out_mlp_W1, out_mlp_b1, out_mlp_W2, out_mlp_b2):
    n = pos.shape[0]
    nb = int_rbf1_W.shape[0]
    h = atom_emb_W.shape[1]

    row = edge_index[0].astype(jnp.int32)
    col = edge_index[1].astype(jnp.int32)
    posx = jnp.asarray(pos[:, 0])
    posy = jnp.asarray(pos[:, 1])
    posz = jnp.asarray(pos[:, 2])

    d2 = _sc_edge_d2(posx, posy, posz, row, col)

    mults = _tc_multipliers(d2[:, None], rbf_freq[None, :], emb_W1,
                            emb_b1[None, :], emb_W2, emb_b2[None, :],
                            int_rbf1_W, int_rbf2_W, out_rbf_W)
    e128 = mults[0]
    Rout = mults[1:2 + nb]
    Rint = mults[2 + nb:]

    zeros128 = jnp.zeros((n, 128), jnp.float32)

    accE = _sc_scatter(e128, col, zeros128)

    emb_pad = jnp.zeros((128, h), jnp.float32).at[:atom_emb_W.shape[0]].set(
        atom_emb_W)
    x, xup = _tc_init(z.astype(jnp.int32)[:, None], emb_pad, accE,
                      int_up_W[0], int_up_b[0][None, :])

    P = jnp.zeros((n, 1), jnp.float32)
    for r in range(nb):
        aggO = _sc_gms(x, Rout[r], row, col, zeros128)
        aggX = _sc_gms(xup, Rint[r], row, col, zeros128)
        upW = int_up_W[r + 1] if r + 1 < nb else int_up_W[r]
        upb = int_up_b[r + 1] if r + 1 < nb else int_up_b[r]
        P, x, xup = _tc_round(x, aggO, aggX, P, out_mlp_W1[r],
                              out_mlp_b1[r][None, :], out_mlp_W2[r],
                              out_mlp_b2[r][None, :], upW, upb[None, :])

    aggO = _sc_gms(x, Rout[nb], row, col, zeros128)
    P = _tc_final(aggO, P, out_mlp_W1[nb], out_mlp_b1[nb][None, :],
                  out_mlp_W2[nb], out_mlp_b2[nb][None, :])
    return P
